# baseline (speedup 1.0000x reference)
import functools

import jax
import jax.numpy as jnp
from jax import lax
from jax.experimental import pallas as pl
from jax.experimental.pallas import tpu as pltpu
from jax.experimental.pallas import tpu_sc as plsc

_F = 26
_V = 100000
_D = 32
_B = 16384
_NW = 32
_BPW = _B // _NW
_CHUNK = 128
_NCH = _BPW // _CHUNK


def _sc_encode(idx_all, tab_flat):
    mesh = plsc.VectorSubcoreMesh(core_axis_name="c", subcore_axis_name="s")

    @functools.partial(
        pl.kernel,
        mesh=mesh,
        out_type=jax.ShapeDtypeStruct((_B, _F, _D), jnp.float32),
        scratch_types=[
            pltpu.VMEM((_NCH, _CHUNK), jnp.int32),
            pltpu.VMEM((_CHUNK, _D), jnp.float32),
            pltpu.SemaphoreType.DMA,
        ],
        compiler_params=pltpu.CompilerParams(use_tc_tiling_on_sc=False),
    )
    def k(idx_hbm, tab_hbm, out_hbm, idx_v, rows_v, sem):
        wid = lax.axis_index("s") * 2 + lax.axis_index("c")
        base = wid * _BPW

        def body(f, carry):
            pltpu.sync_copy(idx_hbm.at[f, wid], idx_v)
            for c in range(_NCH):
                pltpu.async_copy(tab_hbm.at[idx_v.at[c]], rows_v, sem).wait()
                pltpu.sync_copy(
                    rows_v, out_hbm.at[pl.ds(base + c * _CHUNK, _CHUNK), f]
                )
            return carry

        lax.fori_loop(0, _F, body, 0)

    return k(idx_all, tab_flat)


def kernel(sparse_tensors, tables):
    idx = sparse_tensors.astype(jnp.int32)
    offs = (jnp.arange(_F, dtype=jnp.int32) * _V)[:, None]
    gidx = (idx + offs).reshape(_F, _NW, _NCH, _CHUNK)
    tab_flat = tables.reshape(_F * _V, _D)
    out3 = _sc_encode(gidx, tab_flat)
    return out3.reshape(_B, _F * _D)


# speedup vs baseline: 1.0617x; 1.0617x over previous
import functools

import jax
import jax.numpy as jnp
from jax import lax
from jax.experimental import pallas as pl
from jax.experimental.pallas import tpu as pltpu
from jax.experimental.pallas import tpu_sc as plsc

_F = 26
_V = 100000
_D = 32
_B = 16384
_NW = 32
_BPW = _B // _NW
_CHUNK = 128
_NCH = _BPW // _CHUNK
_NBUF = 4


def _sc_encode(idx_all, tab_flat):
    mesh = plsc.VectorSubcoreMesh(core_axis_name="c", subcore_axis_name="s")

    @functools.partial(
        pl.kernel,
        mesh=mesh,
        out_type=jax.ShapeDtypeStruct((_B, _F, _D), jnp.float32),
        scratch_types=[
            pltpu.VMEM((_F * _NCH, _CHUNK), jnp.int32),
            pltpu.VMEM((_NBUF, _BPW, _D), jnp.float32),
            pltpu.SemaphoreType.DMA,
            pltpu.SemaphoreType.DMA,
            pltpu.SemaphoreType.DMA,
            pltpu.SemaphoreType.DMA,
            pltpu.SemaphoreType.DMA,
            pltpu.SemaphoreType.DMA,
            pltpu.SemaphoreType.DMA,
            pltpu.SemaphoreType.DMA,
        ],
        compiler_params=pltpu.CompilerParams(use_tc_tiling_on_sc=False),
    )
    def k(idx_hbm, tab_hbm, out_hbm, idx_v, bufs, *sems):
        gsem = sems[:_NBUF]
        ssem = sems[_NBUF:]
        wid = lax.axis_index("s") * 2 + lax.axis_index("c")
        base = wid * _BPW

        pltpu.sync_copy(idx_hbm.at[wid], idx_v)

        def fire_gathers(f):
            b = f % _NBUF
            return [
                pltpu.async_copy(
                    tab_hbm.at[idx_v.at[f * _NCH + c]],
                    bufs.at[b, pl.ds(c * _CHUNK, _CHUNK)],
                    gsem[b],
                )
                for c in range(_NCH)
            ]

        gh = {}
        sh = {}
        for f in range(_NBUF):
            gh[f] = fire_gathers(f)
        for f in range(_F):
            b = f % _NBUF
            for h in gh.pop(f):
                h.wait()
            sh[f] = pltpu.async_copy(
                bufs.at[b], out_hbm.at[pl.ds(base, _BPW), f], ssem[b]
            )
            nf = f + _NBUF
            if nf < _F:
                sh.pop(f).wait()
                gh[nf] = fire_gathers(nf)
        for f in range(_F - _NBUF, _F):
            sh.pop(f).wait()

    return k(idx_all, tab_flat)


def kernel(sparse_tensors, tables):
    idx = sparse_tensors.astype(jnp.int32)
    offs = (jnp.arange(_F, dtype=jnp.int32) * _V)[:, None]
    gidx = (
        (idx + offs)
        .reshape(_F, _NW, _NCH, _CHUNK)
        .transpose(1, 0, 2, 3)
        .reshape(_NW, _F * _NCH, _CHUNK)
    )
    tab_flat = tables.reshape(_F * _V, _D)
    out3 = _sc_encode(gidx, tab_flat)
    return out3.reshape(_B, _F * _D)
